# Initial kernel scaffold; baseline (speedup 1.0000x reference)
import jax
import jax.numpy as jnp
from jax.experimental import pallas as pl


def kernel(probs, anchor_deltas, img_info):
    raise NotImplementedError("write your pallas kernel here")


# speedup vs baseline: 18.0347x; 18.0347x over previous
import numpy as np
import jax
import jax.numpy as jnp
from jax import lax
from jax.experimental import pallas as pl
from jax.experimental.pallas import tpu as pltpu

_PRE = 2000
_NPAD = 2048
_OUTR = 304
_TH = 0.7
_ROWS, _COLS = 36, 1024
_SLAB = 256


def _gen_anchors():
    ratios = np.array([0.5, 1.0, 2.0])
    scales = np.array([8.0, 16.0, 32.0])
    base = np.array([0.0, 0.0, 15.0, 15.0])
    w = base[2] - base[0] + 1
    h = base[3] - base[1] + 1
    x_ctr = base[0] + 0.5 * (w - 1)
    y_ctr = base[1] + 0.5 * (h - 1)
    size = w * h
    size_ratios = size / ratios
    ws = np.round(np.sqrt(size_ratios))
    hs = np.round(ws * ratios)
    ratio_anchors = np.hstack([
        (x_ctr - 0.5 * (ws - 1))[:, None],
        (y_ctr - 0.5 * (hs - 1))[:, None],
        (x_ctr + 0.5 * (ws - 1))[:, None],
        (y_ctr + 0.5 * (hs - 1))[:, None],
    ])
    out = []
    for a in ratio_anchors:
        w2 = a[2] - a[0] + 1
        h2 = a[3] - a[1] + 1
        xc = a[0] + 0.5 * (w2 - 1)
        yc = a[1] + 0.5 * (h2 - 1)
        ws2 = w2 * scales
        hs2 = h2 * scales
        out.append(np.hstack([
            (xc - 0.5 * (ws2 - 1))[:, None],
            (yc - 0.5 * (hs2 - 1))[:, None],
            (xc + 0.5 * (ws2 - 1))[:, None],
            (yc + 0.5 * (hs2 - 1))[:, None],
        ]))
    return np.vstack(out)


def _build_consts():
    A = _gen_anchors()
    w9 = A[:, 2] - A[:, 0] + 1.0
    h9 = A[:, 3] - A[:, 1] + 1.0
    sx = np.tile(np.arange(64) * 16.0, 64)
    sy = np.repeat(np.arange(64) * 16.0, 64)
    acx = (sx[:, None] + A[None, :, 0] + 0.5 * w9[None, :]).reshape(-1)
    acy = (sy[:, None] + A[None, :, 1] + 0.5 * h9[None, :]).reshape(-1)
    aw = np.broadcast_to(w9[None, :], (4096, 9)).reshape(-1)
    ah = np.broadcast_to(h9[None, :], (4096, 9)).reshape(-1)
    return tuple(
        x.astype(np.float32).reshape(_ROWS, _COLS)
        for x in (aw, ah, acx, acy))


_CONSTS = _build_consts()


def _proposal_body(dx_ref, dy_ref, dw_ref, dh_ref, sc_ref,
                   aw_ref, ah_ref, acx_ref, acy_ref, scal_ref,
                   out_ref, s_scr, p_scr, col_scr):
    f32 = jnp.float32
    i32 = jnp.int32
    clip_w = scal_ref[0, 0]
    clip_h = scal_ref[0, 1]
    min_size = scal_ref[0, 2]

    aw = aw_ref[...]
    ah = ah_ref[...]
    dx = dx_ref[...]
    dy = dy_ref[...]
    dwc = jnp.clip(dw_ref[...], -10.0, 10.0)
    dhc = jnp.clip(dh_ref[...], -10.0, 10.0)
    pcx = dx * aw + acx_ref[...]
    pcy = dy * ah + acy_ref[...]
    pw = jnp.exp(dwc) * aw
    ph = jnp.exp(dhc) * ah
    x1 = pcx - 0.5 * pw
    y1 = pcy - 0.5 * ph
    x2 = pcx + 0.5 * pw - 1.0
    y2 = pcy + 0.5 * ph - 1.0
    x1 = jnp.clip(x1, 0.0, clip_w)
    y1 = jnp.clip(y1, 0.0, clip_h)
    x2 = jnp.clip(x2, 0.0, clip_w)
    y2 = jnp.clip(y2, 0.0, clip_h)
    ws_ = x2 - x1 + 1.0
    hs_ = y2 - y1 + 1.0
    valid = (ws_ >= min_size) & (hs_ >= min_size)
    scm = jnp.where(valid, sc_ref[...], f32(-1e9))

    bits = lax.bitcast_convert_type(scm, i32)
    keys = jnp.where(scm < 0.0, i32(-1), bits)

    def bis_body(_, carry):
        lo, hi = carry
        mid = (lo + hi) // 2
        cnt = jnp.sum((keys >= mid).astype(i32))
        good = cnt >= _PRE
        return jnp.where(good, mid, lo), jnp.where(good, hi, mid)

    T, _ = lax.fori_loop(0, 31, bis_body, (i32(-1), i32(1 << 30)))

    gt = keys > T
    eq = keys == T
    cnt_gt = jnp.sum(gt.astype(f32))

    def ubuild_body(b, _):
        base = b * 128
        li = base + lax.broadcasted_iota(i32, (128, _COLS), 0)
        lj = lax.broadcasted_iota(i32, (128, _COLS), 1)
        s_scr[pl.ds(base, 128), 0:_COLS] = (li < lj).astype(f32)
        return 0

    lax.fori_loop(0, _COLS // 128, ubuild_body, 0)
    ri = lax.broadcasted_iota(i32, (_ROWS, _ROWS), 0)
    rj = lax.broadcasted_iota(i32, (_ROWS, _ROWS), 1)
    l_row = (rj < ri).astype(f32)

    def exprefix(m):
        pe = jnp.dot(m, s_scr[0:_COLS, 0:_COLS], preferred_element_type=f32)
        tot = jnp.sum(m, axis=1, keepdims=True)
        ro = jnp.dot(l_row, tot, preferred_element_type=f32)
        return pe + ro

    pre_eq = exprefix(eq.astype(f32))
    sel = gt | (eq & (cnt_gt + pre_eq < f32(_PRE)))
    sel_f = sel.astype(f32)
    dsts = exprefix(sel_f)

    for p, v in enumerate((x1, y1, x2, y2, scm, dsts, sel_f)):
        p_scr[:, p:p + 1, :] = v.reshape(_ROWS, 1, _COLS)

    def compact_body(c, acc):
        planes = p_scr[pl.ds(c, 1), :, :].reshape(8, _COLS)
        d_c = jnp.transpose(planes[5:6, :])
        s_c = jnp.transpose(planes[6:7, :])
        kio = lax.broadcasted_iota(i32, (_COLS, _NPAD), 1)
        oh = ((kio == d_c.astype(i32)) & (s_c > 0.0)).astype(f32)
        zrow = jnp.zeros((1, _COLS), f32)
        valsT = jnp.concatenate(
            [zrow, planes[0:5, :], zrow, zrow], axis=0)
        return acc + jnp.dot(valsT, oh, preferred_element_type=f32,
                             precision=lax.Precision.HIGHEST)

    compT = lax.fori_loop(0, _ROWS, compact_body,
                          jnp.zeros((8, _NPAD), f32))

    cio = lax.broadcasted_iota(i32, (1, _NPAD), 1)
    s_row = jnp.where(cio < _PRE, compT[5:6, :], f32(-2e9))
    col_scr[:, 0:1] = jnp.transpose(s_row)

    def rank_body(b, acc):
        base = b * _SLAB
        s_cs = col_scr[pl.ds(base, _SLAB), 0:1]
        qg = base + lax.broadcasted_iota(i32, (_SLAB, _NPAD), 0)
        pg = lax.broadcasted_iota(i32, (_SLAB, _NPAD), 1)
        contrib = (s_cs > s_row).astype(f32) + \
                  ((s_cs == s_row) & (qg < pg)).astype(f32)
        return acc + jnp.sum(contrib, axis=0, keepdims=True)

    rank = lax.fori_loop(0, _NPAD // _SLAB, rank_body,
                         jnp.zeros((1, _NPAD), f32))
    col_scr[:, 1:2] = jnp.transpose(rank)

    def perm_body(b, _):
        base = b * _SLAB
        r_cs = col_scr[pl.ds(base, _SLAB), 1:2].astype(i32)
        rio = lax.broadcasted_iota(i32, (_SLAB, _NPAD), 1)
        s_scr[pl.ds(base, _SLAB), :] = (rio == r_cs).astype(f32)
        return 0

    lax.fori_loop(0, _NPAD // _SLAB, perm_body, 0)
    sortT = jnp.dot(compT, s_scr[...], preferred_element_type=f32,
                    precision=lax.Precision.HIGHEST)

    vs_row = jnp.where(cio < _PRE, sortT[5:6, :], f32(-2e9))
    valid_row = vs_row > f32(-1e8)
    bx1r = sortT[1:2, :]
    by1r = sortT[2:3, :]
    bx2r = sortT[3:4, :]
    by2r = sortT[4:5, :]
    area_r = (bx2r - bx1r + 1.0) * (by2r - by1r + 1.0)
    col_scr[:, 2:3] = jnp.transpose(bx1r)
    col_scr[:, 3:4] = jnp.transpose(by1r)
    col_scr[:, 4:5] = jnp.transpose(bx2r)
    col_scr[:, 5:6] = jnp.transpose(by2r)
    col_scr[:, 6:7] = jnp.transpose(area_r)

    def iou_body(b, _):
        base = b * _SLAB
        cols = col_scr[pl.ds(base, _SLAB), :]
        xx1 = jnp.maximum(cols[:, 2:3], bx1r)
        yy1 = jnp.maximum(cols[:, 3:4], by1r)
        xx2 = jnp.minimum(cols[:, 4:5], bx2r)
        yy2 = jnp.minimum(cols[:, 5:6], by2r)
        iw = jnp.maximum(xx2 - xx1 + 1.0, 0.0)
        ih = jnp.maximum(yy2 - yy1 + 1.0, 0.0)
        inter = iw * ih
        iou = inter / (cols[:, 6:7] + area_r - inter)
        ig = base + lax.broadcasted_iota(i32, (_SLAB, _NPAD), 0)
        jg = lax.broadcasted_iota(i32, (_SLAB, _NPAD), 1)
        s_scr[pl.ds(base, _SLAB), :] = \
            ((iou > f32(_TH)) & (jg > ig)).astype(f32)
        return 0

    lax.fori_loop(0, _NPAD // _SLAB, iou_body, 0)

    keep0 = valid_row.astype(f32)

    def nms_cond(c):
        _, changed, it = c
        return changed & (it < _NPAD)

    def nms_body(c):
        k, _, it = c
        m = jnp.dot(k, s_scr[...], preferred_element_type=f32)
        kn = (valid_row & (m == 0.0)).astype(f32)
        return kn, jnp.any(kn != k), it + 1

    keep, _, _ = lax.while_loop(
        nms_cond, nms_body, (keep0, jnp.bool_(True), i32(0)))

    def tri_body(b, _):
        base = b * _SLAB
        pg = base + lax.broadcasted_iota(i32, (_SLAB, _NPAD), 0)
        qg = lax.broadcasted_iota(i32, (_SLAB, _NPAD), 1)
        s_scr[pl.ds(base, _SLAB), :] = (pg < qg).astype(f32)
        return 0

    lax.fori_loop(0, _NPAD // _SLAB, tri_body, 0)
    pc = jnp.dot(keep, s_scr[...], preferred_element_type=f32)
    rank2 = jnp.where(keep > 0.0, pc, f32(1e9))
    r2c = jnp.transpose(rank2).astype(i32)
    fio = lax.broadcasted_iota(i32, (_NPAD, _OUTR), 1)
    FT = (fio == r2c).astype(f32)
    out_ref[...] = jnp.dot(sortT, FT, preferred_element_type=f32,
                           precision=lax.Precision.HIGHEST)


def kernel(probs, anchor_deltas, img_info):
    aw, ah, acx, acy = _CONSTS
    d4 = anchor_deltas[0].reshape(9, 4, 64, 64)
    dx = jnp.transpose(d4[:, 0], (1, 2, 0)).reshape(_ROWS, _COLS)
    dy = jnp.transpose(d4[:, 1], (1, 2, 0)).reshape(_ROWS, _COLS)
    dw = jnp.transpose(d4[:, 2], (1, 2, 0)).reshape(_ROWS, _COLS)
    dh = jnp.transpose(d4[:, 3], (1, 2, 0)).reshape(_ROWS, _COLS)
    sc = jnp.transpose(probs[0, 9:], (1, 2, 0)).reshape(_ROWS, _COLS)
    scal = jnp.stack([img_info[1] - 1.0, img_info[0] - 1.0,
                      16.0 * img_info[2], jnp.float32(0.0)]).reshape(1, 4)

    gt = pl.pallas_call(
        _proposal_body,
        out_shape=jax.ShapeDtypeStruct((8, _OUTR), jnp.float32),
        in_specs=[pl.BlockSpec(memory_space=pltpu.VMEM)] * 9 +
                 [pl.BlockSpec(memory_space=pltpu.SMEM)],
        out_specs=pl.BlockSpec(memory_space=pltpu.VMEM),
        scratch_shapes=[pltpu.VMEM((_NPAD, _NPAD), jnp.float32),
                        pltpu.VMEM((_ROWS, 8, _COLS), jnp.float32),
                        pltpu.VMEM((_NPAD, 8), jnp.float32)],
    )(dx, dy, dw, dh, sc, aw, ah, acx, acy, scal)
    return jnp.transpose(gt[:5, :300])


# speedup vs baseline: 41.0732x; 2.2775x over previous
import numpy as np
import jax
import jax.numpy as jnp
from jax import lax
from jax.experimental import pallas as pl
from jax.experimental.pallas import tpu as pltpu

_PRE = 2000
_NPAD = 2048
_OUTR = 304
_TH = 0.7
_ROWS, _COLS = 36, 1024
_SLAB = 256


def _gen_anchors():
    ratios = np.array([0.5, 1.0, 2.0])
    scales = np.array([8.0, 16.0, 32.0])
    base = np.array([0.0, 0.0, 15.0, 15.0])
    w = base[2] - base[0] + 1
    h = base[3] - base[1] + 1
    x_ctr = base[0] + 0.5 * (w - 1)
    y_ctr = base[1] + 0.5 * (h - 1)
    size = w * h
    size_ratios = size / ratios
    ws = np.round(np.sqrt(size_ratios))
    hs = np.round(ws * ratios)
    ratio_anchors = np.hstack([
        (x_ctr - 0.5 * (ws - 1))[:, None],
        (y_ctr - 0.5 * (hs - 1))[:, None],
        (x_ctr + 0.5 * (ws - 1))[:, None],
        (y_ctr + 0.5 * (hs - 1))[:, None],
    ])
    out = []
    for a in ratio_anchors:
        w2 = a[2] - a[0] + 1
        h2 = a[3] - a[1] + 1
        xc = a[0] + 0.5 * (w2 - 1)
        yc = a[1] + 0.5 * (h2 - 1)
        ws2 = w2 * scales
        hs2 = h2 * scales
        out.append(np.hstack([
            (xc - 0.5 * (ws2 - 1))[:, None],
            (yc - 0.5 * (hs2 - 1))[:, None],
            (xc + 0.5 * (ws2 - 1))[:, None],
            (yc + 0.5 * (hs2 - 1))[:, None],
        ]))
    return np.vstack(out)


def _build_consts():
    A = _gen_anchors()
    w9 = A[:, 2] - A[:, 0] + 1.0
    h9 = A[:, 3] - A[:, 1] + 1.0
    sx = np.tile(np.arange(64) * 16.0, 64)
    sy = np.repeat(np.arange(64) * 16.0, 64)
    acx = (sx[:, None] + A[None, :, 0] + 0.5 * w9[None, :]).reshape(-1)
    acy = (sy[:, None] + A[None, :, 1] + 0.5 * h9[None, :]).reshape(-1)
    aw = np.broadcast_to(w9[None, :], (4096, 9)).reshape(-1)
    ah = np.broadcast_to(h9[None, :], (4096, 9)).reshape(-1)
    return tuple(
        x.astype(np.float32).reshape(_ROWS, _COLS)
        for x in (aw, ah, acx, acy))


_CONSTS = _build_consts()


def _proposal_body(dx_ref, dy_ref, dw_ref, dh_ref, sc_ref,
                   aw_ref, ah_ref, acx_ref, acy_ref, scal_ref,
                   out_ref, s_scr, p_scr, d_scr, col_scr):
    f32 = jnp.float32
    i32 = jnp.int32
    clip_w = scal_ref[0, 0]
    clip_h = scal_ref[0, 1]
    min_size = scal_ref[0, 2]

    aw = aw_ref[...]
    ah = ah_ref[...]
    dx = dx_ref[...]
    dy = dy_ref[...]
    dwc = jnp.clip(dw_ref[...], -10.0, 10.0)
    dhc = jnp.clip(dh_ref[...], -10.0, 10.0)
    pcx = dx * aw + acx_ref[...]
    pcy = dy * ah + acy_ref[...]
    pw = jnp.exp(dwc) * aw
    ph = jnp.exp(dhc) * ah
    x1 = pcx - 0.5 * pw
    y1 = pcy - 0.5 * ph
    x2 = pcx + 0.5 * pw - 1.0
    y2 = pcy + 0.5 * ph - 1.0
    x1 = jnp.clip(x1, 0.0, clip_w)
    y1 = jnp.clip(y1, 0.0, clip_h)
    x2 = jnp.clip(x2, 0.0, clip_w)
    y2 = jnp.clip(y2, 0.0, clip_h)
    ws_ = x2 - x1 + 1.0
    hs_ = y2 - y1 + 1.0
    valid = (ws_ >= min_size) & (hs_ >= min_size)
    scm = jnp.where(valid, sc_ref[...], f32(-1e9))

    bits = lax.bitcast_convert_type(scm, i32)
    keys = jnp.where(scm < 0.0, i32(-1), bits)

    def bis_body(_, carry):
        lo, hi = carry
        mid = (lo + hi) // 2
        cnt = jnp.sum((keys >= mid).astype(i32))
        good = cnt >= _PRE
        return jnp.where(good, mid, lo), jnp.where(good, hi, mid)

    T, _ = lax.fori_loop(0, 31, bis_body, (i32(-1), i32(1 << 30)))

    gt = keys > T
    eq = keys == T
    cnt_gt = jnp.sum(gt.astype(f32))

    bf16 = jnp.bfloat16

    def ubuild_body(b, _):
        base = b * 128
        li = base + lax.broadcasted_iota(i32, (128, _COLS), 0)
        lj = lax.broadcasted_iota(i32, (128, _COLS), 1)
        s_scr[pl.ds(base, 128), 0:_COLS] = (li < lj).astype(bf16)
        return 0

    lax.fori_loop(0, _COLS // 128, ubuild_body, 0)
    ri = lax.broadcasted_iota(i32, (_ROWS, _ROWS), 0)
    rj = lax.broadcasted_iota(i32, (_ROWS, _ROWS), 1)
    l_row = (rj < ri).astype(bf16)

    def exprefix(mb):
        pe = jnp.dot(mb, s_scr[0:_COLS, 0:_COLS], preferred_element_type=f32)
        tot = jnp.sum(mb.astype(f32), axis=1, keepdims=True)
        ro = jnp.dot(l_row, tot.astype(bf16), preferred_element_type=f32)
        return pe + ro, tot

    pre_eq, _ = exprefix(eq.astype(bf16))
    sel = gt | (eq & (cnt_gt + pre_eq < f32(_PRE)))
    dsts, _ = exprefix(sel.astype(bf16))
    dsel = jnp.where(sel, dsts, f32(-1.0))

    zsc = jnp.zeros((_ROWS, 1, _COLS), f32)
    vals8 = (zsc, x1.reshape(_ROWS, 1, _COLS), y1.reshape(_ROWS, 1, _COLS),
             x2.reshape(_ROWS, 1, _COLS), y2.reshape(_ROWS, 1, _COLS),
             scm.reshape(_ROWS, 1, _COLS), zsc, zsc)
    for p, v in enumerate(vals8):
        hi = v.astype(bf16)
        rem = v - hi.astype(f32)
        mid = rem.astype(bf16)
        lo = (rem - mid.astype(f32)).astype(bf16)
        p_scr[:, p:p + 1, :] = hi
        p_scr[:, 8 + p:9 + p, :] = mid
        p_scr[:, 16 + p:17 + p, :] = lo
    d_scr[:, 0:1, :] = dsel.reshape(_ROWS, 1, _COLS)

    def compact_body(c, acc):
        d_c = jnp.transpose(d_scr[pl.ds(c, 1), :, :].reshape(1, _COLS))
        kio = lax.broadcasted_iota(i32, (_COLS, _NPAD), 1)
        oh = (kio == d_c.astype(i32)).astype(bf16)
        valsT = p_scr[pl.ds(c, 1), :, :].reshape(24, _COLS)
        return acc + jnp.dot(valsT, oh, preferred_element_type=f32)

    comp24 = lax.fori_loop(0, _ROWS, compact_body,
                           jnp.zeros((24, _NPAD), f32))
    compT = comp24[0:8, :] + comp24[8:16, :] + comp24[16:24, :]

    def split3(v):
        hi = v.astype(bf16)
        rem = v - hi.astype(f32)
        mid = rem.astype(bf16)
        lo = (rem - mid.astype(f32)).astype(bf16)
        return jnp.concatenate([hi, mid, lo], axis=0)

    cio = lax.broadcasted_iota(i32, (1, _NPAD), 1)
    s_row = jnp.where(cio < _PRE, compT[5:6, :], f32(-2e9))
    col_scr[:, 0:1] = jnp.transpose(s_row)

    def rank_body(b, acc):
        base = b * _SLAB
        s_cs = col_scr[pl.ds(base, _SLAB), 0:1]
        qg = base + lax.broadcasted_iota(i32, (_SLAB, _NPAD), 0)
        pg = lax.broadcasted_iota(i32, (_SLAB, _NPAD), 1)
        contrib = (s_cs > s_row).astype(f32) + \
                  ((s_cs == s_row) & (qg < pg)).astype(f32)
        return acc + jnp.sum(contrib, axis=0, keepdims=True)

    rank = lax.fori_loop(0, _NPAD // _SLAB, rank_body,
                         jnp.zeros((1, _NPAD), f32))
    col_scr[:, 1:2] = jnp.transpose(rank)

    def perm_body(b, _):
        base = b * _SLAB
        r_cs = col_scr[pl.ds(base, _SLAB), 1:2].astype(i32)
        rio = lax.broadcasted_iota(i32, (_SLAB, _NPAD), 1)
        s_scr[pl.ds(base, _SLAB), :] = (rio == r_cs).astype(bf16)
        return 0

    lax.fori_loop(0, _NPAD // _SLAB, perm_body, 0)
    sort24 = jnp.dot(split3(compT), s_scr[...], preferred_element_type=f32)
    sortT = sort24[0:8, :] + sort24[8:16, :] + sort24[16:24, :]

    vs_row = jnp.where(cio < _PRE, sortT[5:6, :], f32(-2e9))
    valid_row = vs_row > f32(-1e8)
    bx1r = sortT[1:2, :]
    by1r = sortT[2:3, :]
    bx2r = sortT[3:4, :]
    by2r = sortT[4:5, :]
    area_r = (bx2r - bx1r + 1.0) * (by2r - by1r + 1.0)
    col_scr[:, 2:3] = jnp.transpose(bx1r)
    col_scr[:, 3:4] = jnp.transpose(by1r)
    col_scr[:, 4:5] = jnp.transpose(bx2r)
    col_scr[:, 5:6] = jnp.transpose(by2r)
    col_scr[:, 6:7] = jnp.transpose(area_r)

    def iou_body(b, _):
        base = b * _SLAB
        cols = col_scr[pl.ds(base, _SLAB), :]
        xx1 = jnp.maximum(cols[:, 2:3], bx1r)
        yy1 = jnp.maximum(cols[:, 3:4], by1r)
        xx2 = jnp.minimum(cols[:, 4:5], bx2r)
        yy2 = jnp.minimum(cols[:, 5:6], by2r)
        iw = jnp.maximum(xx2 - xx1 + 1.0, 0.0)
        ih = jnp.maximum(yy2 - yy1 + 1.0, 0.0)
        inter = iw * ih
        iou = inter / (cols[:, 6:7] + area_r - inter)
        ig = base + lax.broadcasted_iota(i32, (_SLAB, _NPAD), 0)
        jg = lax.broadcasted_iota(i32, (_SLAB, _NPAD), 1)
        s_scr[pl.ds(base, _SLAB), :] = \
            ((iou > f32(_TH)) & (jg > ig)).astype(bf16)
        return 0

    lax.fori_loop(0, _NPAD // _SLAB, iou_body, 0)

    keep0 = valid_row.astype(f32)

    def nms_cond(c):
        _, changed, it = c
        return changed & (it < _NPAD)

    def nms_body(c):
        k, _, it = c
        m = jnp.dot(k.astype(bf16), s_scr[...], preferred_element_type=f32)
        kn = (valid_row & (m == 0.0)).astype(f32)
        return kn, jnp.any(kn != k), it + 1

    keep, _, _ = lax.while_loop(
        nms_cond, nms_body, (keep0, jnp.bool_(True), i32(0)))

    def tri_body(b, _):
        base = b * _SLAB
        pg = base + lax.broadcasted_iota(i32, (_SLAB, _NPAD), 0)
        qg = lax.broadcasted_iota(i32, (_SLAB, _NPAD), 1)
        s_scr[pl.ds(base, _SLAB), :] = (pg < qg).astype(bf16)
        return 0

    lax.fori_loop(0, _NPAD // _SLAB, tri_body, 0)
    pc = jnp.dot(keep.astype(bf16), s_scr[...], preferred_element_type=f32)
    rank2 = jnp.where(keep > 0.0, pc, f32(1e9))
    r2c = jnp.transpose(rank2).astype(i32)
    fio = lax.broadcasted_iota(i32, (_NPAD, _OUTR), 1)
    FT = (fio == r2c).astype(bf16)
    o24 = jnp.dot(split3(sortT), FT, preferred_element_type=f32)
    out_ref[...] = o24[0:8, :] + o24[8:16, :] + o24[16:24, :]


def kernel(probs, anchor_deltas, img_info):
    aw, ah, acx, acy = _CONSTS
    d4 = anchor_deltas[0].reshape(9, 4, 64, 64)
    dx = jnp.transpose(d4[:, 0], (1, 2, 0)).reshape(_ROWS, _COLS)
    dy = jnp.transpose(d4[:, 1], (1, 2, 0)).reshape(_ROWS, _COLS)
    dw = jnp.transpose(d4[:, 2], (1, 2, 0)).reshape(_ROWS, _COLS)
    dh = jnp.transpose(d4[:, 3], (1, 2, 0)).reshape(_ROWS, _COLS)
    sc = jnp.transpose(probs[0, 9:], (1, 2, 0)).reshape(_ROWS, _COLS)
    scal = jnp.stack([img_info[1] - 1.0, img_info[0] - 1.0,
                      16.0 * img_info[2], jnp.float32(0.0)]).reshape(1, 4)

    gt = pl.pallas_call(
        _proposal_body,
        out_shape=jax.ShapeDtypeStruct((8, _OUTR), jnp.float32),
        in_specs=[pl.BlockSpec(memory_space=pltpu.VMEM)] * 9 +
                 [pl.BlockSpec(memory_space=pltpu.SMEM)],
        out_specs=pl.BlockSpec(memory_space=pltpu.VMEM),
        scratch_shapes=[pltpu.VMEM((_NPAD, _NPAD), jnp.bfloat16),
                        pltpu.VMEM((_ROWS, 24, _COLS), jnp.bfloat16),
                        pltpu.VMEM((_ROWS, 1, _COLS), jnp.float32),
                        pltpu.VMEM((_NPAD, 8), jnp.float32)],
    )(dx, dy, dw, dh, sc, aw, ah, acx, acy, scal)
    return jnp.transpose(gt[:5, :300])


# speedup vs baseline: 44.7790x; 1.0902x over previous
import numpy as np
import jax
import jax.numpy as jnp
from jax import lax
from jax.experimental import pallas as pl
from jax.experimental.pallas import tpu as pltpu

_PRE = 2000
_NPAD = 2048
_OUTR = 304
_TH = 0.7
_ROWS, _COLS = 36, 1024
_SLAB = 256


def _gen_anchors():
    ratios = np.array([0.5, 1.0, 2.0])
    scales = np.array([8.0, 16.0, 32.0])
    base = np.array([0.0, 0.0, 15.0, 15.0])
    w = base[2] - base[0] + 1
    h = base[3] - base[1] + 1
    x_ctr = base[0] + 0.5 * (w - 1)
    y_ctr = base[1] + 0.5 * (h - 1)
    size = w * h
    size_ratios = size / ratios
    ws = np.round(np.sqrt(size_ratios))
    hs = np.round(ws * ratios)
    ratio_anchors = np.hstack([
        (x_ctr - 0.5 * (ws - 1))[:, None],
        (y_ctr - 0.5 * (hs - 1))[:, None],
        (x_ctr + 0.5 * (ws - 1))[:, None],
        (y_ctr + 0.5 * (hs - 1))[:, None],
    ])
    out = []
    for a in ratio_anchors:
        w2 = a[2] - a[0] + 1
        h2 = a[3] - a[1] + 1
        xc = a[0] + 0.5 * (w2 - 1)
        yc = a[1] + 0.5 * (h2 - 1)
        ws2 = w2 * scales
        hs2 = h2 * scales
        out.append(np.hstack([
            (xc - 0.5 * (ws2 - 1))[:, None],
            (yc - 0.5 * (hs2 - 1))[:, None],
            (xc + 0.5 * (ws2 - 1))[:, None],
            (yc + 0.5 * (hs2 - 1))[:, None],
        ]))
    return np.vstack(out)


def _build_consts():
    A = _gen_anchors()
    w9 = A[:, 2] - A[:, 0] + 1.0
    h9 = A[:, 3] - A[:, 1] + 1.0
    sx = np.tile(np.arange(64) * 16.0, 64)
    sy = np.repeat(np.arange(64) * 16.0, 64)
    acx = (sx[:, None] + A[None, :, 0] + 0.5 * w9[None, :]).reshape(-1)
    acy = (sy[:, None] + A[None, :, 1] + 0.5 * h9[None, :]).reshape(-1)
    aw = np.broadcast_to(w9[None, :], (4096, 9)).reshape(-1)
    ah = np.broadcast_to(h9[None, :], (4096, 9)).reshape(-1)
    return tuple(
        x.astype(np.float32).reshape(_ROWS, _COLS)
        for x in (aw, ah, acx, acy))


_CONSTS = _build_consts()


def _proposal_body(dall_ref, sc_ref,
                   aw_ref, ah_ref, acx_ref, acy_ref, scal_ref,
                   out_ref, s_scr, p_scr, d_scr, col_scr):
    f32 = jnp.float32
    i32 = jnp.int32
    clip_w = scal_ref[0, 1] - 1.0
    clip_h = scal_ref[0, 0] - 1.0
    min_size = 16.0 * scal_ref[0, 2]

    aw = aw_ref[...]
    ah = ah_ref[...]
    dx = dall_ref[0]
    dy = dall_ref[1]
    dwc = jnp.clip(dall_ref[2], -10.0, 10.0)
    dhc = jnp.clip(dall_ref[3], -10.0, 10.0)
    pcx = dx * aw + acx_ref[...]
    pcy = dy * ah + acy_ref[...]
    pw = jnp.exp(dwc) * aw
    ph = jnp.exp(dhc) * ah
    x1 = pcx - 0.5 * pw
    y1 = pcy - 0.5 * ph
    x2 = pcx + 0.5 * pw - 1.0
    y2 = pcy + 0.5 * ph - 1.0
    x1 = jnp.clip(x1, 0.0, clip_w)
    y1 = jnp.clip(y1, 0.0, clip_h)
    x2 = jnp.clip(x2, 0.0, clip_w)
    y2 = jnp.clip(y2, 0.0, clip_h)
    ws_ = x2 - x1 + 1.0
    hs_ = y2 - y1 + 1.0
    valid = (ws_ >= min_size) & (hs_ >= min_size)
    scm = jnp.where(valid, sc_ref[...], f32(-1e9))

    bits = lax.bitcast_convert_type(scm, i32)
    keys = jnp.where(scm < 0.0, i32(-1), bits)

    def bis_body(_, carry):
        lo, hi = carry
        mid = (lo + hi) // 2
        cnt = jnp.sum((keys >= mid).astype(i32))
        good = cnt >= _PRE
        return jnp.where(good, mid, lo), jnp.where(good, hi, mid)

    T, _ = lax.fori_loop(0, 31, bis_body, (i32(-1), i32(1 << 30)))

    gt = keys > T
    eq = keys == T
    cnt_gt = jnp.sum(gt.astype(f32))

    bf16 = jnp.bfloat16

    def ubuild_body(b, _):
        base = b * 128
        li = base + lax.broadcasted_iota(i32, (128, _COLS), 0)
        lj = lax.broadcasted_iota(i32, (128, _COLS), 1)
        s_scr[pl.ds(base, 128), 0:_COLS] = (li < lj).astype(bf16)
        return 0

    lax.fori_loop(0, _COLS // 128, ubuild_body, 0)
    ri = lax.broadcasted_iota(i32, (_ROWS, _ROWS), 0)
    rj = lax.broadcasted_iota(i32, (_ROWS, _ROWS), 1)
    l_row = (rj < ri).astype(bf16)

    def exprefix(mb):
        pe = jnp.dot(mb, s_scr[0:_COLS, 0:_COLS], preferred_element_type=f32)
        tot = jnp.sum(mb.astype(f32), axis=1, keepdims=True)
        ro = jnp.dot(l_row, tot.astype(bf16), preferred_element_type=f32)
        return pe + ro, tot

    pre_eq, _ = exprefix(eq.astype(bf16))
    sel = gt | (eq & (cnt_gt + pre_eq < f32(_PRE)))
    dsts, _ = exprefix(sel.astype(bf16))
    dsel = jnp.where(sel, dsts, f32(-1.0))

    zb = jnp.zeros((_ROWS, 1, _COLS), bf16)
    for p in (0, 6, 7):
        p_scr[:, p:p + 1, :] = zb
        p_scr[:, 8 + p:9 + p, :] = zb
        p_scr[:, 16 + p:17 + p, :] = zb
    for p, vv in ((1, x1), (2, y1), (3, x2), (4, y2), (5, scm)):
        v = vv.reshape(_ROWS, 1, _COLS)
        hi = v.astype(bf16)
        rem = v - hi.astype(f32)
        mid = rem.astype(bf16)
        lo = (rem - mid.astype(f32)).astype(bf16)
        p_scr[:, p:p + 1, :] = hi
        p_scr[:, 8 + p:9 + p, :] = mid
        p_scr[:, 16 + p:17 + p, :] = lo
    d_scr[:, 0:1, :] = dsel.reshape(_ROWS, 1, _COLS)

    def compact_body(c, acc):
        d_c = jnp.transpose(d_scr[pl.ds(c, 1), :, :].reshape(1, _COLS))
        kio = lax.broadcasted_iota(i32, (_COLS, _NPAD), 1)
        oh = (kio == d_c.astype(i32)).astype(bf16)
        valsT = p_scr[pl.ds(c, 1), :, :].reshape(24, _COLS)
        return acc + jnp.dot(valsT, oh, preferred_element_type=f32)

    comp24 = lax.fori_loop(0, _ROWS, compact_body,
                           jnp.zeros((24, _NPAD), f32))
    compT = comp24[0:8, :] + comp24[8:16, :] + comp24[16:24, :]

    def split3(v):
        hi = v.astype(bf16)
        rem = v - hi.astype(f32)
        mid = rem.astype(bf16)
        lo = (rem - mid.astype(f32)).astype(bf16)
        return jnp.concatenate([hi, mid, lo], axis=0)

    cio = lax.broadcasted_iota(i32, (1, _NPAD), 1)
    s_row = jnp.where(cio < _PRE, compT[5:6, :], f32(-2e9))
    col_scr[:, 0:1] = jnp.transpose(s_row)

    def rank_body(b, acc):
        base = b * _SLAB
        s_cs = col_scr[pl.ds(base, _SLAB), 0:1]
        qg = base + lax.broadcasted_iota(i32, (_SLAB, _NPAD), 0)
        pg = lax.broadcasted_iota(i32, (_SLAB, _NPAD), 1)
        contrib = (s_cs > s_row).astype(f32) + \
                  ((s_cs == s_row) & (qg < pg)).astype(f32)
        return acc + jnp.sum(contrib, axis=0, keepdims=True)

    rank = lax.fori_loop(0, _NPAD // _SLAB, rank_body,
                         jnp.zeros((1, _NPAD), f32))
    col_scr[:, 1:2] = jnp.transpose(rank)

    def perm_body(b, _):
        base = b * _SLAB
        r_cs = col_scr[pl.ds(base, _SLAB), 1:2].astype(i32)
        rio = lax.broadcasted_iota(i32, (_SLAB, _NPAD), 1)
        s_scr[pl.ds(base, _SLAB), :] = (rio == r_cs).astype(bf16)
        return 0

    lax.fori_loop(0, _NPAD // _SLAB, perm_body, 0)
    sort24 = jnp.dot(split3(compT), s_scr[...], preferred_element_type=f32)
    sortT = sort24[0:8, :] + sort24[8:16, :] + sort24[16:24, :]

    vs_row = jnp.where(cio < _PRE, sortT[5:6, :], f32(-2e9))
    valid_row = vs_row > f32(-1e8)
    bx1r = sortT[1:2, :]
    by1r = sortT[2:3, :]
    bx2r = sortT[3:4, :]
    by2r = sortT[4:5, :]
    area_r = (bx2r - bx1r + 1.0) * (by2r - by1r + 1.0)
    col_scr[:, 2:3] = jnp.transpose(bx1r)
    col_scr[:, 3:4] = jnp.transpose(by1r)
    col_scr[:, 4:5] = jnp.transpose(bx2r)
    col_scr[:, 5:6] = jnp.transpose(by2r)
    col_scr[:, 6:7] = jnp.transpose(area_r)

    for b in range(_NPAD // _SLAB):
        base = b * _SLAB
        ncol = _NPAD - base
        cols = col_scr[base:base + _SLAB, :]
        xx1 = jnp.maximum(cols[:, 2:3], bx1r[:, base:])
        yy1 = jnp.maximum(cols[:, 3:4], by1r[:, base:])
        xx2 = jnp.minimum(cols[:, 4:5], bx2r[:, base:])
        yy2 = jnp.minimum(cols[:, 5:6], by2r[:, base:])
        iw = jnp.maximum(xx2 - xx1 + 1.0, 0.0)
        ih = jnp.maximum(yy2 - yy1 + 1.0, 0.0)
        inter = iw * ih
        iou = inter / (cols[:, 6:7] + area_r[:, base:] - inter)
        ig = lax.broadcasted_iota(i32, (_SLAB, ncol), 0)
        jg = lax.broadcasted_iota(i32, (_SLAB, ncol), 1)
        s_scr[base:base + _SLAB, base:] = \
            ((iou > f32(_TH)) & (jg > ig)).astype(bf16)

    keep_parts = []
    m_acc = jnp.zeros((1, _NPAD), f32)
    for b in range(_NPAD // _SLAB):
        base = b * _SLAB
        sbb = s_scr[base:base + _SLAB, base:base + _SLAB]
        vfree = valid_row[:, base:base + _SLAB] & \
            (m_acc[:, base:base + _SLAB] == 0.0)

        def blk_cond(c):
            _, changed, it = c
            return changed & (it < _SLAB)

        def blk_body(c, sbb=sbb, vfree=vfree):
            k, _, it = c
            mb = jnp.dot(k.astype(bf16), sbb, preferred_element_type=f32)
            kn = (vfree & (mb == 0.0)).astype(f32)
            return kn, jnp.any(kn != k), it + 1

        kb, _, _ = lax.while_loop(
            blk_cond, blk_body,
            (vfree.astype(f32), jnp.bool_(True), i32(0)))
        m_acc = m_acc + jnp.dot(kb.astype(bf16), s_scr[base:base + _SLAB, :],
                                preferred_element_type=f32)
        keep_parts.append(kb)
    keep = jnp.concatenate(keep_parts, axis=1)

    def tri_body(b, _):
        base = b * _SLAB
        pg = base + lax.broadcasted_iota(i32, (_SLAB, _NPAD), 0)
        qg = lax.broadcasted_iota(i32, (_SLAB, _NPAD), 1)
        s_scr[pl.ds(base, _SLAB), :] = (pg < qg).astype(bf16)
        return 0

    lax.fori_loop(0, _NPAD // _SLAB, tri_body, 0)
    pc = jnp.dot(keep.astype(bf16), s_scr[...], preferred_element_type=f32)
    rank2 = jnp.where(keep > 0.0, pc, f32(1e9))
    r2c = jnp.transpose(rank2).astype(i32)
    fio = lax.broadcasted_iota(i32, (_NPAD, _OUTR), 1)
    FT = (fio == r2c).astype(bf16)
    o24 = jnp.dot(split3(sortT), FT, preferred_element_type=f32)
    out_ref[...] = o24[0:8, :] + o24[8:16, :] + o24[16:24, :]


def kernel(probs, anchor_deltas, img_info):
    aw, ah, acx, acy = _CONSTS
    d4 = anchor_deltas[0].reshape(9, 4, 64, 64)
    dall = jnp.transpose(d4, (1, 2, 3, 0)).reshape(4, _ROWS, _COLS)
    sc = jnp.transpose(probs[0, 9:], (1, 2, 0)).reshape(_ROWS, _COLS)
    scal = img_info.reshape(1, 3)

    gt = pl.pallas_call(
        _proposal_body,
        out_shape=jax.ShapeDtypeStruct((8, _OUTR), jnp.float32),
        in_specs=[pl.BlockSpec(memory_space=pltpu.VMEM)] * 6 +
                 [pl.BlockSpec(memory_space=pltpu.SMEM)],
        out_specs=pl.BlockSpec(memory_space=pltpu.VMEM),
        scratch_shapes=[pltpu.VMEM((_NPAD, _NPAD), jnp.bfloat16),
                        pltpu.VMEM((_ROWS, 24, _COLS), jnp.bfloat16),
                        pltpu.VMEM((_ROWS, 1, _COLS), jnp.float32),
                        pltpu.VMEM((_NPAD, 8), jnp.float32)],
    )(dall, sc, aw, ah, acx, acy, scal)
    return jnp.transpose(gt[:5, :300])
